# Initial kernel scaffold; baseline (speedup 1.0000x reference)
import functools

import jax
import jax.numpy as jnp
from jax import lax
from jax.experimental import pallas as pl
from jax.experimental.pallas import tpu as pltpu
from jax.experimental.pallas import tpu_sc as plsc

N = 10000
E = 320000
D_IN = 128
D_HID = 200
D_OUT = 8
D_PAD = 16

NC = 2
NS = 16
NW = NC * NS
E_PER_W = E // NW
CHUNK = 80
N_CHUNKS = E_PER_W // CHUNK
ROWS_PER_SUB = N // NS

_mesh = plsc.VectorSubcoreMesh(core_axis_name="c", subcore_axis_name="s")


def _make_sc_agg(d):
  @functools.partial(
      pl.kernel,
      out_type=jax.ShapeDtypeStruct((NC, N, d), jnp.float32),
      mesh=_mesh,
      scratch_types=[
          pltpu.VMEM((CHUNK,), jnp.int32),
          pltpu.VMEM((CHUNK,), jnp.int32),
          pltpu.VMEM((CHUNK,), jnp.float32),
          pltpu.VMEM((CHUNK, d), jnp.float32),
          pltpu.VMEM_SHARED((N, d), jnp.float32),
          pltpu.SemaphoreType.DMA,
      ],
  )
  def agg(vals_hbm, row_hbm, col_hbm, ew_hbm, zeros_hbm, out_hbm,
          rowbuf, colbuf, ewbuf, gbuf, acc, sem):
    c = lax.axis_index("c")
    s = lax.axis_index("s")
    base = (c * NS + s) * E_PER_W

    @pl.when(s == 0)
    def _():
      pltpu.sync_copy(zeros_hbm, acc)

    plsc.subcore_barrier()

    @pl.loop(0, N_CHUNKS)
    def _(i):
      off = base + i * CHUNK
      pltpu.sync_copy(row_hbm.at[pl.ds(off, CHUNK)], rowbuf)
      pltpu.sync_copy(col_hbm.at[pl.ds(off, CHUNK)], colbuf)
      pltpu.sync_copy(ew_hbm.at[pl.ds(off, CHUNK)], ewbuf)
      pltpu.async_copy(vals_hbm.at[rowbuf], gbuf, sem).wait()

      @pl.loop(0, CHUNK)
      def _(e):
        sv = plsc.load_gather(ewbuf, [jnp.zeros((16,), jnp.int32) + e])
        sv = sv.reshape(1, 16)
        for j in range(d // 16):
          slc = (pl.ds(e, 1), pl.ds(j * 16, 16))
          gbuf[slc] = gbuf[slc] * sv

      pltpu.sync_copy(gbuf, acc.at[colbuf], add=True)

    plsc.subcore_barrier()
    pltpu.sync_copy(acc.at[pl.ds(s * ROWS_PER_SUB, ROWS_PER_SUB)],
                    out_hbm.at[c].at[pl.ds(s * ROWS_PER_SUB, ROWS_PER_SUB)])

  return agg


_sc_agg128 = _make_sc_agg(D_IN)
_sc_agg16 = _make_sc_agg(D_PAD)


@functools.partial(
    pl.kernel,
    out_type=jax.ShapeDtypeStruct((NC, N, D_PAD), jnp.float32),
    mesh=_mesh,
    scratch_types=[
        pltpu.VMEM((CHUNK,), jnp.int32),
        pltpu.VMEM((CHUNK,), jnp.float32),
        pltpu.VMEM((CHUNK, D_PAD), jnp.float32),
        pltpu.VMEM_SHARED((N, D_PAD), jnp.float32),
        pltpu.SemaphoreType.DMA,
    ],
)
def _sc_deg(col_hbm, ew_hbm, zeros_hbm, out_hbm, colbuf, ewbuf, gbuf, acc, sem):
  c = lax.axis_index("c")
  s = lax.axis_index("s")
  base = (c * NS + s) * E_PER_W

  @pl.when(s == 0)
  def _():
    pltpu.sync_copy(zeros_hbm, acc)

  plsc.subcore_barrier()

  @pl.loop(0, N_CHUNKS)
  def _(i):
    off = base + i * CHUNK
    pltpu.sync_copy(col_hbm.at[pl.ds(off, CHUNK)], colbuf)
    pltpu.sync_copy(ew_hbm.at[pl.ds(off, CHUNK)], ewbuf)

    @pl.loop(0, CHUNK)
    def _(e):
      sv = plsc.load_gather(ewbuf, [jnp.zeros((16,), jnp.int32) + e])
      gbuf[pl.ds(e, 1), :] = sv.reshape(1, 16)

    pltpu.sync_copy(gbuf, acc.at[colbuf], add=True)

  plsc.subcore_barrier()
  pltpu.sync_copy(acc.at[pl.ds(s * ROWS_PER_SUB, ROWS_PER_SUB)],
                  out_hbm.at[c].at[pl.ds(s * ROWS_PER_SUB, ROWS_PER_SUB)])


_BLK = 1000


def _dis_from(deg0, deg1):
  deg = deg0[:, :1] + deg1[:, :1] + 1.0
  return jnp.where(deg > 0, lax.rsqrt(deg), 0.0)


def _tc_prescale_body(deg0_ref, deg1_ref, x_ref, xt_ref):
  dis = _dis_from(deg0_ref[...], deg1_ref[...])
  xt_ref[...] = x_ref[...] * dis


def _tc_mid_body(deg0_ref, deg1_ref, a0_ref, a1_ref, xt_ref, w1_ref, b1_ref,
                 w2_ref, pt_ref):
  dis = _dis_from(deg0_ref[...], deg1_ref[...])
  a = (a0_ref[...] + a1_ref[...] + xt_ref[...]) * dis
  h = jnp.dot(a, w1_ref[...], preferred_element_type=jnp.float32) + b1_ref[...]
  h = jnp.maximum(h, 0.0)
  p = jnp.dot(h, w2_ref[...], preferred_element_type=jnp.float32)
  pt_ref[...] = p * dis


def _tc_final_body(deg0_ref, deg1_ref, a0_ref, a1_ref, pt_ref, b2_ref, out_ref):
  dis = _dis_from(deg0_ref[...], deg1_ref[...])
  t = (a0_ref[...] + a1_ref[...] + pt_ref[...]) * dis
  out_ref[...] = t[:, :D_OUT] + b2_ref[...]


def _nblock(width):
  return pl.BlockSpec((_BLK, width), lambda i: (i, 0))


def _full(shape):
  return pl.BlockSpec(shape, lambda i: tuple(0 for _ in shape))


def kernel(x, edge_index, edge_attr, W1, b1, W2, b2):
  row = edge_index[0]
  col = edge_index[1]
  ew = edge_attr
  z128 = jnp.zeros((N, D_IN), jnp.float32)
  z16 = jnp.zeros((N, D_PAD), jnp.float32)
  W2p = jnp.pad(W2, ((0, 0), (0, D_PAD - D_OUT)))
  b1r = b1.reshape(1, D_HID)
  b2r = b2.reshape(1, D_OUT)

  deg = _sc_deg(col, ew, z16)
  deg0, deg1 = deg[0], deg[1]

  xt = pl.pallas_call(
      _tc_prescale_body,
      grid=(N // _BLK,),
      in_specs=[_nblock(D_PAD), _nblock(D_PAD), _nblock(D_IN)],
      out_specs=_nblock(D_IN),
      out_shape=jax.ShapeDtypeStruct((N, D_IN), jnp.float32),
  )(deg0, deg1, x)

  acc1 = _sc_agg128(xt, row, col, ew, z128)

  pt = pl.pallas_call(
      _tc_mid_body,
      grid=(N // _BLK,),
      in_specs=[_nblock(D_PAD), _nblock(D_PAD), _nblock(D_IN), _nblock(D_IN),
                _nblock(D_IN), _full((D_IN, D_HID)), _full((1, D_HID)),
                _full((D_HID, D_PAD))],
      out_specs=_nblock(D_PAD),
      out_shape=jax.ShapeDtypeStruct((N, D_PAD), jnp.float32),
  )(deg0, deg1, acc1[0], acc1[1], xt, W1, b1r, W2p)

  acc2 = _sc_agg16(pt, row, col, ew, z16)

  out = pl.pallas_call(
      _tc_final_body,
      grid=(N // _BLK,),
      in_specs=[_nblock(D_PAD), _nblock(D_PAD), _nblock(D_PAD), _nblock(D_PAD),
                _nblock(D_PAD), _full((1, D_OUT))],
      out_specs=_nblock(D_OUT),
      out_shape=jax.ShapeDtypeStruct((N, D_OUT), jnp.float32),
  )(deg0, deg1, acc2[0], acc2[1], pt, b2r)

  return out


# speedup vs baseline: 8.9466x; 8.9466x over previous
import dataclasses
import functools

import jax
import jax.numpy as jnp
from jax import lax
from jax.experimental import pallas as pl
from jax.experimental.pallas import tpu as pltpu
from jax.experimental.pallas import tpu_sc as plsc

N = 10000
E = 320000
D_IN = 128
D_HID = 200
D_OUT = 8

NC = 2
NS = 16
E_PER_W = E // (NC * NS)
CHUNK = 80
N_CHUNKS = E_PER_W // CHUNK
WB_ROWS = 1000
WB_SUBS = N // WB_ROWS

_mesh = plsc.VectorSubcoreMesh(core_axis_name="c", subcore_axis_name="s")

_sc_cp = pltpu.CompilerParams()
if "needs_layout_passes" in pltpu.CompilerParams.__dataclass_fields__:
  _sc_cp = dataclasses.replace(_sc_cp, needs_layout_passes=False)


def _make_sc_agg(nblk):
  @functools.partial(
      pl.kernel,
      out_type=jax.ShapeDtypeStruct((NC, N, D_IN), jnp.float32),
      mesh=_mesh,
      compiler_params=_sc_cp,
      scratch_types=[
          pltpu.VMEM((CHUNK,), jnp.int32),
          pltpu.VMEM((CHUNK,), jnp.int32),
          pltpu.VMEM((CHUNK,), jnp.float32),
          pltpu.VMEM((CHUNK, D_IN), jnp.float32),
          pltpu.VMEM_SHARED((N, D_IN), jnp.float32),
          pltpu.SemaphoreType.DMA,
      ],
  )
  def agg(vals_hbm, row_hbm, col_hbm, ew_hbm, zeros_hbm, out_hbm,
          rowbuf, colbuf, ewbuf, gbuf, acc, sem):
    c = lax.axis_index("c")
    s = lax.axis_index("s")
    base = (c * NS + s) * E_PER_W

    @pl.when(s == 0)
    def _():
      pltpu.sync_copy(zeros_hbm, acc)

    plsc.subcore_barrier()

    @pl.loop(0, N_CHUNKS)
    def _(i):
      off = base + i * CHUNK
      pltpu.sync_copy(row_hbm.at[pl.ds(off, CHUNK)], rowbuf)
      pltpu.sync_copy(col_hbm.at[pl.ds(off, CHUNK)], colbuf)
      pltpu.sync_copy(ew_hbm.at[pl.ds(off, CHUNK)], ewbuf)
      pltpu.async_copy(vals_hbm.at[rowbuf], gbuf, sem).wait()

      @pl.loop(0, CHUNK)
      def _(e):
        sv = plsc.load_gather(ewbuf, [jnp.zeros((16,), jnp.int32) + e])
        for j in range(nblk):
          slc = (e, pl.ds(j * 16, 16))
          gbuf[slc] = gbuf[slc] * sv

      pltpu.sync_copy(gbuf, acc.at[colbuf], add=True)

    plsc.subcore_barrier()

    @pl.when(s < WB_SUBS)
    def _():
      pltpu.sync_copy(acc.at[pl.ds(s * WB_ROWS, WB_ROWS)],
                      out_hbm.at[c].at[pl.ds(s * WB_ROWS, WB_ROWS)])

  return agg


_sc_agg128 = _make_sc_agg(D_IN // 16)
_sc_agg16 = _make_sc_agg(1)


@functools.partial(
    pl.kernel,
    out_type=jax.ShapeDtypeStruct((NC, N, D_IN), jnp.float32),
    mesh=_mesh,
    compiler_params=_sc_cp,
    scratch_types=[
        pltpu.VMEM((CHUNK,), jnp.int32),
        pltpu.VMEM((CHUNK,), jnp.float32),
        pltpu.VMEM((CHUNK, D_IN), jnp.float32),
        pltpu.VMEM_SHARED((N, D_IN), jnp.float32),
        pltpu.SemaphoreType.DMA,
    ],
)
def _sc_deg(col_hbm, ew_hbm, zeros_hbm, out_hbm, colbuf, ewbuf, gbuf, acc, sem):
  c = lax.axis_index("c")
  s = lax.axis_index("s")
  base = (c * NS + s) * E_PER_W

  zv = jnp.zeros((16,), jnp.float32)

  @pl.loop(0, CHUNK)
  def _(e):
    for j in range(D_IN // 16):
      gbuf[e, pl.ds(j * 16, 16)] = zv

  @pl.when(s == 0)
  def _():
    pltpu.sync_copy(zeros_hbm, acc)

  plsc.subcore_barrier()

  @pl.loop(0, N_CHUNKS)
  def _(i):
    off = base + i * CHUNK
    pltpu.sync_copy(col_hbm.at[pl.ds(off, CHUNK)], colbuf)
    pltpu.sync_copy(ew_hbm.at[pl.ds(off, CHUNK)], ewbuf)

    @pl.loop(0, CHUNK)
    def _(e):
      sv = plsc.load_gather(ewbuf, [jnp.zeros((16,), jnp.int32) + e])
      gbuf[e, pl.ds(0, 16)] = sv

    pltpu.sync_copy(gbuf, acc.at[colbuf], add=True)

  plsc.subcore_barrier()

  @pl.when(s < WB_SUBS)
  def _():
    pltpu.sync_copy(acc.at[pl.ds(s * WB_ROWS, WB_ROWS)],
                    out_hbm.at[c].at[pl.ds(s * WB_ROWS, WB_ROWS)])


_BLK = 1000


def _dis_from(deg0, deg1):
  deg = deg0[:, :1] + deg1[:, :1] + 1.0
  return jnp.where(deg > 0, lax.rsqrt(deg), 0.0)


def _tc_prescale_body(deg0_ref, deg1_ref, x_ref, xt_ref):
  dis = _dis_from(deg0_ref[...], deg1_ref[...])
  xt_ref[...] = x_ref[...] * dis


def _tc_mid_body(deg0_ref, deg1_ref, a0_ref, a1_ref, xt_ref, w1_ref, b1_ref,
                 w2_ref, pt_ref):
  dis = _dis_from(deg0_ref[...], deg1_ref[...])
  a = (a0_ref[...] + a1_ref[...] + xt_ref[...]) * dis
  h = jnp.dot(a, w1_ref[...], preferred_element_type=jnp.float32) + b1_ref[...]
  h = jnp.maximum(h, 0.0)
  p = jnp.dot(h, w2_ref[...], preferred_element_type=jnp.float32)
  pt_ref[...] = p * dis


def _tc_final_body(deg0_ref, deg1_ref, a0_ref, a1_ref, pt_ref, b2_ref, out_ref):
  dis = _dis_from(deg0_ref[...], deg1_ref[...])
  t = (a0_ref[...] + a1_ref[...] + pt_ref[...]) * dis
  out_ref[...] = t[:, :D_OUT] + b2_ref[...]


def _nblock(width):
  return pl.BlockSpec((_BLK, width), lambda i: (i, 0))


def _full(shape):
  return pl.BlockSpec(shape, lambda i: tuple(0 for _ in shape))


def kernel(x, edge_index, edge_attr, W1, b1, W2, b2):
  row = edge_index[0]
  col = edge_index[1]
  ew = edge_attr
  z128 = jnp.zeros((N, D_IN), jnp.float32)
  W2p = jnp.pad(W2, ((0, 0), (0, D_IN - D_OUT)))
  b1r = b1.reshape(1, D_HID)
  b2r = b2.reshape(1, D_OUT)

  deg = _sc_deg(col, ew, z128)
  deg0, deg1 = deg[0], deg[1]

  xt = pl.pallas_call(
      _tc_prescale_body,
      grid=(N // _BLK,),
      in_specs=[_nblock(D_IN), _nblock(D_IN), _nblock(D_IN)],
      out_specs=_nblock(D_IN),
      out_shape=jax.ShapeDtypeStruct((N, D_IN), jnp.float32),
  )(deg0, deg1, x)

  acc1 = _sc_agg128(xt, row, col, ew, z128)

  pt = pl.pallas_call(
      _tc_mid_body,
      grid=(N // _BLK,),
      in_specs=[_nblock(D_IN), _nblock(D_IN), _nblock(D_IN), _nblock(D_IN),
                _nblock(D_IN), _full((D_IN, D_HID)), _full((1, D_HID)),
                _full((D_HID, D_IN))],
      out_specs=_nblock(D_IN),
      out_shape=jax.ShapeDtypeStruct((N, D_IN), jnp.float32),
  )(deg0, deg1, acc1[0], acc1[1], xt, W1, b1r, W2p)

  acc2 = _sc_agg16(pt, row, col, ew, z128)

  out = pl.pallas_call(
      _tc_final_body,
      grid=(N // _BLK,),
      in_specs=[_nblock(D_IN), _nblock(D_IN), _nblock(D_IN), _nblock(D_IN),
                _nblock(D_IN), _full((1, D_OUT))],
      out_specs=_nblock(D_OUT),
      out_shape=jax.ShapeDtypeStruct((N, D_OUT), jnp.float32),
  )(deg0, deg1, acc2[0], acc2[1], pt, b2r)

  return out


# speedup vs baseline: 21.8661x; 2.4441x over previous
import dataclasses
import functools

import jax
import jax.numpy as jnp
from jax import lax
from jax.experimental import pallas as pl
from jax.experimental.pallas import tpu as pltpu
from jax.experimental.pallas import tpu_sc as plsc

N = 10000
E = 320000
D_IN = 128
D_HID = 200
D_OUT = 8

NC = 2
NS = 16
E_PER_W = E // (NC * NS)
CHUNK = 80
NCH = E_PER_W // CHUNK
NBUF = 4
WB_ROWS = 1000
WB_SUBS = N // WB_ROWS

_mesh = plsc.VectorSubcoreMesh(core_axis_name="c", subcore_axis_name="s")

_sc_cp = pltpu.CompilerParams()
if "needs_layout_passes" in pltpu.CompilerParams.__dataclass_fields__:
  _sc_cp = dataclasses.replace(_sc_cp, needs_layout_passes=False)


def _make_sc_agg(nblk):
  NB = 3

  @functools.partial(
      pl.kernel,
      out_type=jax.ShapeDtypeStruct((NC, N, D_IN), jnp.float32),
      mesh=_mesh,
      compiler_params=_sc_cp,
      scratch_types=[
          pltpu.VMEM((NB * CHUNK,), jnp.int32),
          pltpu.VMEM((NB, CHUNK), jnp.int32),
          pltpu.VMEM((NB * CHUNK,), jnp.float32),
          pltpu.VMEM((NB, CHUNK, D_IN), jnp.float32),
          pltpu.VMEM_SHARED((N, D_IN), jnp.float32),
          pltpu.SemaphoreType.DMA((NB,)),
          pltpu.SemaphoreType.DMA((NB,)),
          pltpu.SemaphoreType.DMA((NB,)),
      ],
  )
  def agg(vals_hbm, row_hbm, col_hbm, ew_hbm, zeros_hbm, out_hbm,
          rowbuf, colbuf, ewbuf, gbuf, acc, semi, semg, sems):
    c = lax.axis_index("c")
    s = lax.axis_index("s")
    ebase = (c * NS + s) * E_PER_W

    def idx_issue(j, q):
      off = ebase + j * CHUNK
      pltpu.async_copy(row_hbm.at[pl.ds(off, CHUNK)],
                       rowbuf.at[pl.ds(q * CHUNK, CHUNK)], semi.at[q])
      pltpu.async_copy(col_hbm.at[pl.ds(off, CHUNK)], colbuf.at[q],
                       semi.at[q])
      pltpu.async_copy(ew_hbm.at[pl.ds(off, CHUNK)],
                       ewbuf.at[pl.ds(q * CHUNK, CHUNK)], semi.at[q])

    def idx_wait(q):
      for _ in range(3):
        pltpu.make_async_copy(col_hbm.at[pl.ds(ebase, CHUNK)], colbuf.at[q],
                              semi.at[q]).wait()

    def g_issue(j, q):
      pltpu.async_copy(
          vals_hbm.at[rowbuf.at[pl.ds(q * CHUNK, CHUNK)]], gbuf.at[q],
          semg.at[q])

    def g_wait(q):
      pltpu.make_async_copy(vals_hbm.at[rowbuf.at[pl.ds(q * CHUNK, CHUNK)]],
                            gbuf.at[q], semg.at[q]).wait()

    def s_issue(q):
      pltpu.async_copy(gbuf.at[q], acc.at[colbuf.at[q]], sems.at[q], add=True)

    def s_wait(q):
      pltpu.make_async_copy(gbuf.at[q], acc.at[colbuf.at[q]], sems.at[q]).wait()

    def scale(b):
      gb = gbuf.at[b]

      @pl.loop(0, CHUNK, step=16)
      def _(g):
        for t in range(16):
          e = g + t
          sv = plsc.load_gather(ewbuf, [lax.broadcast(b * CHUNK + e, (16,))])
          for j in range(nblk):
            gb[e, pl.ds(j * 16, 16)] = gb[e, pl.ds(j * 16, 16)] * sv

    @pl.when(s == 0)
    def _():
      pltpu.sync_copy(zeros_hbm, acc)

    plsc.subcore_barrier()

    idx_issue(0, 0)
    idx_issue(1, 1)
    idx_wait(0)
    g_issue(0, 0)

    @pl.loop(0, NCH - 2, step=NB)
    def _(i0):
      for t in range(NB):
        i = i0 + t
        b = t
        bn = (t + 1) % NB
        bp = (t + 2) % NB
        g_wait(b)
        if t == 0:
          @pl.when(i0 >= 1)
          def _():
            s_wait(bp)
        else:
          s_wait(bp)
        idx_wait(bn)
        g_issue(i + 1, bn)
        idx_issue(i + 2, bp)
        scale(b)
        s_issue(b)

    g_wait(0)
    s_wait(2)
    idx_wait(1)
    g_issue(NCH - 1, 1)
    scale(0)
    s_issue(0)

    g_wait(1)
    s_wait(0)
    scale(1)
    s_issue(1)
    s_wait(1)

    plsc.subcore_barrier()

    @pl.when(s < WB_SUBS)
    def _():
      pltpu.sync_copy(acc.at[pl.ds(s * WB_ROWS, WB_ROWS)],
                      out_hbm.at[c].at[pl.ds(s * WB_ROWS, WB_ROWS)])

  return agg


_sc_agg128 = _make_sc_agg(D_IN // 16)
_sc_agg16 = _make_sc_agg(1)


@functools.partial(
    pl.kernel,
    out_type=jax.ShapeDtypeStruct((NC, N, D_IN), jnp.float32),
    mesh=_mesh,
    compiler_params=_sc_cp,
    scratch_types=[
        pltpu.VMEM((NCH, CHUNK), jnp.int32),
        pltpu.VMEM((E_PER_W,), jnp.float32),
        pltpu.VMEM((2, CHUNK, D_IN), jnp.float32),
        pltpu.VMEM_SHARED((N, D_IN), jnp.float32),
        pltpu.SemaphoreType.DMA,
        pltpu.SemaphoreType.DMA((2,)),
    ],
)
def _sc_deg(col_hbm, ew_hbm, zeros_hbm, out_hbm, colbuf, ewbuf, gbuf, acc,
            sem0, sems):
  c = lax.axis_index("c")
  s = lax.axis_index("s")
  ebase = (c * NS + s) * E_PER_W

  pltpu.async_copy(ew_hbm.at[pl.ds(ebase, E_PER_W)], ewbuf, sem0)

  @pl.loop(0, NCH)
  def _(i):
    pltpu.async_copy(col_hbm.at[pl.ds(ebase + i * CHUNK, CHUNK)],
                     colbuf.at[i], sem0)

  zv = jnp.zeros((16,), jnp.float32)
  for b in range(2):
    gb = gbuf.at[b]

    @pl.loop(0, CHUNK)
    def _(e):
      for j in range(D_IN // 16):
        gb[e, pl.ds(j * 16, 16)] = zv

  @pl.when(s == 0)
  def _():
    pltpu.sync_copy(zeros_hbm, acc)

  for _i in range(2):
    pltpu.make_async_copy(ew_hbm.at[pl.ds(ebase, E_PER_W)], ewbuf, sem0).wait()

  plsc.subcore_barrier()

  def fill(i, b):
    gb = gbuf.at[b]

    @pl.loop(0, CHUNK, step=16)
    def _(g):
      for t in range(16):
        e = g + t
        sv = plsc.load_gather(ewbuf, [lax.broadcast(i * CHUNK + e, (16,))])
        gb[e, pl.ds(0, 16)] = sv

  def s_issue(i, b):
    pltpu.async_copy(gbuf.at[b], acc.at[colbuf.at[i]], sems.at[b], add=True)

  def s_wait(b):
    pltpu.make_async_copy(gbuf.at[b], acc.at[colbuf.at[0]], sems.at[b]).wait()

  @pl.loop(0, NCH - 1, step=2)
  def _(i0):
    for t in range(2):
      i = i0 + t
      b = t

      @pl.when(i0 >= 2)
      def _():
        s_wait(b)

      fill(i, b)
      s_issue(i, b)

  s_wait(0)
  fill(NCH - 1, 0)
  s_issue(NCH - 1, 0)
  s_wait(1)
  s_wait(0)

  plsc.subcore_barrier()

  @pl.when(s < WB_SUBS)
  def _():
    pltpu.sync_copy(acc.at[pl.ds(s * WB_ROWS, WB_ROWS)],
                    out_hbm.at[c].at[pl.ds(s * WB_ROWS, WB_ROWS)])


_BLK = 1000


def _dis_from(deg0, deg1):
  deg = deg0[:, :1] + deg1[:, :1] + 1.0
  return jnp.where(deg > 0, lax.rsqrt(deg), 0.0)


def _tc_prescale_body(deg0_ref, deg1_ref, x_ref, xt_ref):
  dis = _dis_from(deg0_ref[...], deg1_ref[...])
  xt_ref[...] = x_ref[...] * dis


def _tc_mid_body(deg0_ref, deg1_ref, a0_ref, a1_ref, xt_ref, w1_ref, b1_ref,
                 w2_ref, pt_ref):
  dis = _dis_from(deg0_ref[...], deg1_ref[...])
  a = (a0_ref[...] + a1_ref[...] + xt_ref[...]) * dis
  h = jnp.dot(a, w1_ref[...], preferred_element_type=jnp.float32) + b1_ref[...]
  h = jnp.maximum(h, 0.0)
  p = jnp.dot(h, w2_ref[...], preferred_element_type=jnp.float32)
  pt_ref[...] = p * dis


def _tc_final_body(deg0_ref, deg1_ref, a0_ref, a1_ref, pt_ref, b2_ref, out_ref):
  dis = _dis_from(deg0_ref[...], deg1_ref[...])
  t = (a0_ref[...] + a1_ref[...] + pt_ref[...]) * dis
  out_ref[...] = t[:, :D_OUT] + b2_ref[...]


def _nblock(width):
  return pl.BlockSpec((_BLK, width), lambda i: (i, 0))


def _full(shape):
  return pl.BlockSpec(shape, lambda i: tuple(0 for _ in shape))


def kernel(x, edge_index, edge_attr, W1, b1, W2, b2):
  row = edge_index[0]
  col = edge_index[1]
  ew = edge_attr
  z128 = jnp.zeros((N, D_IN), jnp.float32)
  W2p = jnp.pad(W2, ((0, 0), (0, D_IN - D_OUT)))
  b1r = b1.reshape(1, D_HID)
  b2r = b2.reshape(1, D_OUT)

  deg = _sc_deg(col, ew, z128)
  deg0, deg1 = deg[0], deg[1]

  xt = pl.pallas_call(
      _tc_prescale_body,
      grid=(N // _BLK,),
      in_specs=[_nblock(D_IN), _nblock(D_IN), _nblock(D_IN)],
      out_specs=_nblock(D_IN),
      out_shape=jax.ShapeDtypeStruct((N, D_IN), jnp.float32),
  )(deg0, deg1, x)

  acc1 = _sc_agg128(xt, row, col, ew, z128)

  pt = pl.pallas_call(
      _tc_mid_body,
      grid=(N // _BLK,),
      in_specs=[_nblock(D_IN), _nblock(D_IN), _nblock(D_IN), _nblock(D_IN),
                _nblock(D_IN), _full((D_IN, D_HID)), _full((1, D_HID)),
                _full((D_HID, D_IN))],
      out_specs=_nblock(D_IN),
      out_shape=jax.ShapeDtypeStruct((N, D_IN), jnp.float32),
  )(deg0, deg1, acc1[0], acc1[1], xt, W1, b1r, W2p)

  acc2 = _sc_agg16(pt, row, col, ew, z128)

  out = pl.pallas_call(
      _tc_final_body,
      grid=(N // _BLK,),
      in_specs=[_nblock(D_IN), _nblock(D_IN), _nblock(D_IN), _nblock(D_IN),
                _nblock(D_IN), _full((1, D_OUT))],
      out_specs=_nblock(D_OUT),
      out_shape=jax.ShapeDtypeStruct((N, D_OUT), jnp.float32),
  )(deg0, deg1, acc2[0], acc2[1], pt, b2r)

  return out
